# Initial kernel scaffold; baseline (speedup 1.0000x reference)
import jax
import jax.numpy as jnp
from jax.experimental import pallas as pl


def kernel(indices, adj_mat, embed, W1, b1, W2, b2):
    raise NotImplementedError("write your pallas kernel here")


# speedup vs baseline: 1.9120x; 1.9120x over previous
import functools

import jax
import jax.numpy as jnp
from jax import lax
from jax.experimental import pallas as pl
from jax.experimental.pallas import tpu as pltpu
from jax.experimental.pallas import tpu_sc as plsc

N = 8192
EMB = 128
NUM_WORKERS = 32
CHUNK = 128
N_CHUNK_ROWS = N // CHUNK
CHUNKS_PER_W = N_CHUNK_ROWS // NUM_WORKERS

BM = 256
GRID = N // BM


def _sc_gather_body(table_hbm, idx_hbm, out_hbm, idx_v, rows_v, sem):
    wid = lax.axis_index("s") * 2 + lax.axis_index("c")
    base = wid * CHUNKS_PER_W
    pltpu.sync_copy(idx_hbm.at[pl.ds(base, CHUNKS_PER_W)], idx_v)
    copies = [
        pltpu.async_copy(table_hbm.at[idx_v.at[j]], rows_v.at[j], sem)
        for j in range(CHUNKS_PER_W)
    ]
    for c in copies:
        c.wait()
    pltpu.sync_copy(rows_v, out_hbm.at[pl.ds(base, CHUNKS_PER_W)])


def _sc_gather(embed, idx2d):
    mesh = plsc.VectorSubcoreMesh(core_axis_name="c", subcore_axis_name="s")
    run = pl.kernel(
        _sc_gather_body,
        out_type=jax.ShapeDtypeStruct((N_CHUNK_ROWS, CHUNK, EMB), jnp.float32),
        mesh=mesh,
        scratch_types=[
            pltpu.VMEM((CHUNKS_PER_W, CHUNK), jnp.int32),
            pltpu.VMEM((CHUNKS_PER_W, CHUNK, EMB), jnp.float32),
            pltpu.SemaphoreType.DMA,
        ],
    )
    return run(embed, idx2d)


def _tc_body(x0_ref, adj_ref, w1_ref, b1_ref, w2_ref, b2_ref, out_ref,
             x1_ref, y_ref, cs_ref):
    i = pl.program_id(0)

    @pl.when(i == 0)
    def _gc1():
        x1_ref[...] = jnp.maximum(
            jnp.dot(x0_ref[...], w1_ref[...],
                    preferred_element_type=jnp.float32) + b1_ref[...], 0.0)

    blk = adj_ref[...]
    y_ref[pl.ds(i * BM, BM), :] = jnp.dot(
        blk, x1_ref[...], preferred_element_type=jnp.float32)
    part = jnp.sum(blk, axis=0, keepdims=True)

    @pl.when(i == 0)
    def _cs_init():
        cs_ref[...] = part

    @pl.when(i > 0)
    def _cs_acc():
        cs_ref[...] = cs_ref[...] + part

    @pl.when(i == GRID - 1)
    def _readout():
        x2 = jnp.maximum(
            jnp.dot(y_ref[...], w2_ref[...],
                    preferred_element_type=jnp.float32) + b2_ref[...], 0.0)
        out_ref[...] = jnp.dot(cs_ref[...], x2,
                               preferred_element_type=jnp.float32)


def _tc_fused(x0, adj, W1, b1, W2, b2):
    return pl.pallas_call(
        _tc_body,
        grid=(GRID,),
        in_specs=[
            pl.BlockSpec((N, EMB), lambda i: (0, 0)),
            pl.BlockSpec((BM, N), lambda i: (i, 0)),
            pl.BlockSpec((EMB, EMB), lambda i: (0, 0)),
            pl.BlockSpec((1, EMB), lambda i: (0, 0)),
            pl.BlockSpec((EMB, EMB), lambda i: (0, 0)),
            pl.BlockSpec((1, EMB), lambda i: (0, 0)),
        ],
        out_specs=pl.BlockSpec((1, EMB), lambda i: (0, 0)),
        out_shape=jax.ShapeDtypeStruct((1, EMB), jnp.float32),
        scratch_shapes=[
            pltpu.VMEM((N, EMB), jnp.float32),
            pltpu.VMEM((N, EMB), jnp.float32),
            pltpu.VMEM((1, N), jnp.float32),
        ],
    )(x0, adj, W1, b1, W2, b2)


@jax.jit
def kernel(indices, adj_mat, embed, W1, b1, W2, b2):
    idx2d = indices.astype(jnp.int32).reshape(N_CHUNK_ROWS, CHUNK)
    x0 = _sc_gather(embed, idx2d).reshape(N, EMB)
    out = _tc_fused(x0, adj_mat, W1, b1.reshape(1, EMB), W2,
                    b2.reshape(1, EMB))
    return out.reshape(EMB)
